# baseline (speedup 1.0000x reference)
import functools

import jax
import jax.numpy as jnp
from jax import lax
from jax.experimental import pallas as pl
from jax.experimental.pallas import tpu as pltpu
from jax.experimental.pallas import tpu_sc as plsc

_B = 16384
_F = 32
_N = 1000000
_L = 16

_INFO = plsc.get_sparse_core_info()
_NC = _INFO.num_cores
_NS = _INFO.num_subcores
_NW = _NC * _NS
_BPW = _B // _NW
_CHUNK = 128
_NCHUNK = _BPW // _CHUNK

_mesh = plsc.VectorSubcoreMesh(core_axis_name="c", subcore_axis_name="s")


@functools.partial(
    pl.kernel,
    out_type=jax.ShapeDtypeStruct((_B,), jnp.float32),
    mesh=_mesh,
    compiler_params=pltpu.CompilerParams(needs_layout_passes=False,
                                         use_tc_tiling_on_sc=False),
    scratch_types=[
        pltpu.VMEM((_NCHUNK, _CHUNK), jnp.int32),
        pltpu.VMEM((_NCHUNK, _CHUNK), jnp.int32),
        pltpu.VMEM((_F, _BPW), jnp.float32),
        pltpu.VMEM((_F, _BPW), jnp.float32),
        pltpu.VMEM((_BPW,), jnp.float32),
        pltpu.VMEM((_BPW,), jnp.float32),
        pltpu.VMEM((_BPW,), jnp.float32),
        pltpu.SemaphoreType.DMA,
    ],
)
def _mf_kernel(uid_hbm, iid_hbm, pf_hbm, qf_hbm, ub_hbm, ib_hbm, out_hbm,
               uidx_v, iidx_v, pv, qv, ub_v, ib_v, out_v, sem):
    wid = lax.axis_index("s") * _NC + lax.axis_index("c")
    base = wid * _BPW

    for j in range(_NCHUNK):
        pltpu.sync_copy(uid_hbm.at[pl.ds(base + j * _CHUNK, _CHUNK)],
                        uidx_v.at[j])
        pltpu.sync_copy(iid_hbm.at[pl.ds(base + j * _CHUNK, _CHUNK)],
                        iidx_v.at[j])

    for j in range(_NCHUNK):
        cs = pl.ds(j * _CHUNK, _CHUNK)
        pltpu.async_copy(ub_hbm.at[uidx_v.at[j]], ub_v.at[cs], sem)
        pltpu.async_copy(ib_hbm.at[iidx_v.at[j]], ib_v.at[cs], sem)

    for f in range(_F):
        fp = pf_hbm.at[pl.ds(f * _N, _N)]
        fq = qf_hbm.at[pl.ds(f * _N, _N)]
        for j in range(_NCHUNK):
            cs = pl.ds(j * _CHUNK, _CHUNK)
            pltpu.async_copy(fp.at[uidx_v.at[j]], pv.at[f].at[cs], sem)
            pltpu.async_copy(fq.at[iidx_v.at[j]], qv.at[f].at[cs], sem)

    for f in range(_F):
        pltpu.make_async_copy(pf_hbm.at[pl.ds(0, _BPW)], pv.at[f], sem).wait()
        pltpu.make_async_copy(qf_hbm.at[pl.ds(0, _BPW)], qv.at[f], sem).wait()
    pltpu.make_async_copy(ub_hbm.at[pl.ds(0, _BPW)], ub_v, sem).wait()
    pltpu.make_async_copy(ib_hbm.at[pl.ds(0, _BPW)], ib_v, sem).wait()

    def block(b, carry):
        sl = pl.ds(b * _L, _L)
        acc = ub_v[sl] + ib_v[sl]
        for f in range(_F):
            acc = acc + pv[f, sl] * qv[f, sl]
        out_v[sl] = acc
        return carry

    lax.fori_loop(0, _BPW // _L, block, 0)

    pltpu.sync_copy(out_v, out_hbm.at[pl.ds(base, _BPW)])


def kernel(user_id, item_id, P, Q, user_bias, item_bias):
    return _mf_kernel(user_id.astype(jnp.int32), item_id.astype(jnp.int32),
                      P.T.reshape(-1), Q.T.reshape(-1),
                      user_bias.reshape(-1), item_bias.reshape(-1))


# speedup vs baseline: 1.0013x; 1.0013x over previous
import functools

import jax
import jax.numpy as jnp
from jax import lax
from jax.experimental import pallas as pl
from jax.experimental.pallas import tpu as pltpu
from jax.experimental.pallas import tpu_sc as plsc

_B = 16384
_F = 32
_N = 1000000
_L = 16

_INFO = plsc.get_sparse_core_info()
_NC = _INFO.num_cores
_NS = _INFO.num_subcores
_NW = _NC * _NS
_BPW = _B // _NW
_CHUNK = 128
_NCHUNK = _BPW // _CHUNK

_mesh = plsc.VectorSubcoreMesh(core_axis_name="c", subcore_axis_name="s")


@functools.partial(
    pl.kernel,
    out_type=jax.ShapeDtypeStruct((_B,), jnp.float32),
    mesh=_mesh,
    compiler_params=pltpu.CompilerParams(needs_layout_passes=False,
                                         use_tc_tiling_on_sc=False),
    scratch_types=[
        pltpu.VMEM((_NCHUNK, _CHUNK), jnp.int32),
        pltpu.VMEM((_NCHUNK, _CHUNK), jnp.int32),
        pltpu.VMEM((_F, _BPW), jnp.float32),
        pltpu.VMEM((_F, _BPW), jnp.float32),
        pltpu.VMEM((_BPW,), jnp.float32),
        pltpu.VMEM((_BPW,), jnp.float32),
        pltpu.VMEM((_BPW,), jnp.float32),
        pltpu.SemaphoreType.DMA,
    ],
)
def _mf_kernel(uid_hbm, iid_hbm, pf_hbm, qf_hbm, ub_hbm, ib_hbm, out_hbm,
               uidx_v, iidx_v, pv, qv, ub_v, ib_v, out_v, sem):
    wid = lax.axis_index("s") * _NC + lax.axis_index("c")
    base = wid * _BPW

    for j in range(_NCHUNK):
        pltpu.sync_copy(uid_hbm.at[pl.ds(base + j * _CHUNK, _CHUNK)],
                        uidx_v.at[j])
        pltpu.sync_copy(iid_hbm.at[pl.ds(base + j * _CHUNK, _CHUNK)],
                        iidx_v.at[j])

    for j in range(_NCHUNK):
        cs = pl.ds(j * _CHUNK, _CHUNK)
        pltpu.async_copy(ub_hbm.at[uidx_v.at[j]], ub_v.at[cs], sem)
        pltpu.async_copy(ib_hbm.at[iidx_v.at[j]], ib_v.at[cs], sem)

    for f in range(_F):
        fp = pf_hbm.at[f]
        fq = qf_hbm.at[f]
        for j in range(_NCHUNK):
            cs = pl.ds(j * _CHUNK, _CHUNK)
            pltpu.async_copy(fp.at[uidx_v.at[j]], pv.at[f].at[cs], sem)
            pltpu.async_copy(fq.at[iidx_v.at[j]], qv.at[f].at[cs], sem)

    for f in range(_F):
        pltpu.make_async_copy(pf_hbm.at[0].at[pl.ds(0, _BPW)], pv.at[f],
                              sem).wait()
        pltpu.make_async_copy(qf_hbm.at[0].at[pl.ds(0, _BPW)], qv.at[f],
                              sem).wait()
    pltpu.make_async_copy(ub_hbm.at[pl.ds(0, _BPW)], ub_v, sem).wait()
    pltpu.make_async_copy(ib_hbm.at[pl.ds(0, _BPW)], ib_v, sem).wait()

    def block(b, carry):
        sl = pl.ds(b * _L, _L)
        acc = ub_v[sl] + ib_v[sl]
        for f in range(_F):
            acc = acc + pv[f, sl] * qv[f, sl]
        out_v[sl] = acc
        return carry

    lax.fori_loop(0, _BPW // _L, block, 0)

    pltpu.sync_copy(out_v, out_hbm.at[pl.ds(base, _BPW)])


def kernel(user_id, item_id, P, Q, user_bias, item_bias):
    return _mf_kernel(user_id.astype(jnp.int32), item_id.astype(jnp.int32),
                      P.T, Q.T,
                      user_bias.reshape(-1), item_bias.reshape(-1))


# speedup vs baseline: 24.6964x; 24.6655x over previous
import functools

import jax
import jax.numpy as jnp
from jax import lax
from jax.experimental import pallas as pl
from jax.experimental.pallas import tpu as pltpu
from jax.experimental.pallas import tpu_sc as plsc

_B = 16384
_F = 32
_N = 1000000
_L = 16

_INFO = plsc.get_sparse_core_info()
_NC = _INFO.num_cores
_NS = _INFO.num_subcores
_NW = _NC * _NS
_BPW = _B // _NW
_NSLOT = 8
_CHUNK = 128

_mesh = plsc.VectorSubcoreMesh(core_axis_name="c", subcore_axis_name="s")


@functools.partial(
    pl.kernel,
    out_type=jax.ShapeDtypeStruct((_B,), jnp.float32),
    mesh=_mesh,
    compiler_params=pltpu.CompilerParams(needs_layout_passes=False,
                                         use_tc_tiling_on_sc=True),
    scratch_types=[
        pltpu.VMEM((_BPW + _L,), jnp.int32),
        pltpu.VMEM((_BPW + _L,), jnp.int32),
        pltpu.VMEM((_NSLOT, _F, 128), jnp.float32),
        pltpu.VMEM((_NSLOT, _F, 128), jnp.float32),
        pltpu.VMEM((_BPW,), jnp.float32),
        pltpu.SemaphoreType.DMA,
    ],
)
def _dot_kernel(uid_hbm, iid_hbm, pt_hbm, qt_hbm, out_hbm,
                uidx_s, iidx_s, pp, qp, out_v, sem):
    wid = lax.axis_index("s") * _NC + lax.axis_index("c")
    base = wid * _BPW

    pltpu.sync_copy(uid_hbm.at[pl.ds(base, _BPW)], uidx_s.at[pl.ds(0, _BPW)])
    pltpu.sync_copy(iid_hbm.at[pl.ds(base, _BPW)], iidx_s.at[pl.ds(0, _BPW)])

    def fire(r, slot):
        u = uidx_s[pl.ds(r, _L)][0]
        i = iidx_s[pl.ds(r, _L)][0]
        ua = pl.multiple_of((u >> 7) << 7, 128)
        ia = pl.multiple_of((i >> 7) << 7, 128)
        pltpu.async_copy(pt_hbm.at[:, pl.ds(ua, 128)], pp.at[slot], sem)
        pltpu.async_copy(qt_hbm.at[:, pl.ds(ia, 128)], qp.at[slot], sem)

    for s in range(_NSLOT):
        fire(s, s)

    rows_lo = lax.iota(jnp.int32, _L)
    rows_hi = rows_lo + _L
    lane = lax.iota(jnp.int32, _L)

    def group(g, acc):
        for s in range(_NSLOT):
            r = g * _NSLOT + s
            pltpu.make_async_copy(pt_hbm.at[:, pl.ds(0, 128)], pp.at[s],
                                  sem).wait()
            pltpu.make_async_copy(qt_hbm.at[:, pl.ds(0, 128)], qp.at[s],
                                  sem).wait()
            uv = uidx_s[pl.ds(r, _L)]
            iv = iidx_s[pl.ds(r, _L)]
            cu = jnp.full((_L,), uv[0] & 127, jnp.int32)
            ci = jnp.full((_L,), iv[0] & 127, jnp.int32)
            p_lo = plsc.load_gather(pp.at[s], [rows_lo, cu])
            p_hi = plsc.load_gather(pp.at[s], [rows_hi, cu])
            q_lo = plsc.load_gather(qp.at[s], [rows_lo, ci])
            q_hi = plsc.load_gather(qp.at[s], [rows_hi, ci])
            t = p_lo * q_lo + p_hi * q_hi
            d = jnp.sum(t, axis=0)
            acc = jnp.where(lane == (r % _L), acc + d, acc)
            @pl.when(r + _NSLOT < _BPW)
            def _():
                fire(r + _NSLOT, s)
        @pl.when(g % 2 == 1)
        def _():
            out_v[pl.ds((g // 2) * _L, _L)] = acc
        return jnp.where(g % 2 == 1, jnp.zeros((_L,), jnp.float32), acc)

    lax.fori_loop(0, _BPW // _NSLOT, group, jnp.zeros((_L,), jnp.float32))

    pltpu.sync_copy(out_v, out_hbm.at[pl.ds(base, _BPW)])


@functools.partial(
    pl.kernel,
    out_type=jax.ShapeDtypeStruct((_B,), jnp.float32),
    mesh=_mesh,
    compiler_params=pltpu.CompilerParams(needs_layout_passes=False,
                                         use_tc_tiling_on_sc=False),
    scratch_types=[
        pltpu.VMEM((_BPW // _CHUNK, _CHUNK), jnp.int32),
        pltpu.VMEM((_BPW // _CHUNK, _CHUNK), jnp.int32),
        pltpu.VMEM((_BPW,), jnp.float32),
        pltpu.VMEM((_BPW,), jnp.float32),
        pltpu.VMEM((_BPW,), jnp.float32),
        pltpu.SemaphoreType.DMA,
    ],
)
def _bias_kernel(dot_hbm, uid_hbm, iid_hbm, ub_hbm, ib_hbm, out_hbm,
                 uidx_v, iidx_v, dot_v, ub_v, ib_v, sem):
    wid = lax.axis_index("s") * _NC + lax.axis_index("c")
    base = wid * _BPW
    nchunk = _BPW // _CHUNK

    for j in range(nchunk):
        pltpu.sync_copy(uid_hbm.at[pl.ds(base + j * _CHUNK, _CHUNK)],
                        uidx_v.at[j])
        pltpu.sync_copy(iid_hbm.at[pl.ds(base + j * _CHUNK, _CHUNK)],
                        iidx_v.at[j])
    for j in range(nchunk):
        cs = pl.ds(j * _CHUNK, _CHUNK)
        pltpu.async_copy(ub_hbm.at[uidx_v.at[j]], ub_v.at[cs], sem)
        pltpu.async_copy(ib_hbm.at[iidx_v.at[j]], ib_v.at[cs], sem)
    pltpu.sync_copy(dot_hbm.at[pl.ds(base, _BPW)], dot_v)
    pltpu.make_async_copy(ub_hbm.at[pl.ds(0, _BPW)], ub_v, sem).wait()
    pltpu.make_async_copy(ib_hbm.at[pl.ds(0, _BPW)], ib_v, sem).wait()

    def block(b, carry):
        sl = pl.ds(b * _L, _L)
        dot_v[sl] = dot_v[sl] + ub_v[sl] + ib_v[sl]
        return carry

    lax.fori_loop(0, _BPW // _L, block, 0)
    pltpu.sync_copy(dot_v, out_hbm.at[pl.ds(base, _BPW)])


def kernel(user_id, item_id, P, Q, user_bias, item_bias):
    uid = user_id.astype(jnp.int32)
    iid = item_id.astype(jnp.int32)
    dots = _dot_kernel(uid, iid, P.T, Q.T)
    return _bias_kernel(dots, uid, iid,
                        user_bias.reshape(-1), item_bias.reshape(-1))
